# Initial kernel scaffold; baseline (speedup 1.0000x reference)
import jax
import jax.numpy as jnp
from jax.experimental import pallas as pl


def kernel(x, edge_index, W1, b1, W2, b2):
    raise NotImplementedError("write your pallas kernel here")


# speedup vs baseline: 2.4733x; 2.4733x over previous
import jax
import jax.numpy as jnp
from jax.experimental import pallas as pl

N_NODES = 10000
D = 128
T_STEPS = 8
NB = 1000


def _mm_scale_kernel(h_ref, dis_ref, w_ref, o_ref):
    h = h_ref[...]
    dis = dis_ref[...]
    hw = jnp.dot(h, w_ref[...], preferred_element_type=jnp.float32)
    o_ref[...] = hw * dis


def _mm_scaled(h2d, dis, W):
    R = h2d.shape[0]
    grid = (R // NB,)
    return pl.pallas_call(
        _mm_scale_kernel,
        grid=grid,
        in_specs=[
            pl.BlockSpec((NB, D), lambda n: (n, 0)),
            pl.BlockSpec((NB, 1), lambda n: (n, 0)),
            pl.BlockSpec((D, D), lambda n: (0, 0)),
        ],
        out_specs=pl.BlockSpec((NB, D), lambda n: (n, 0)),
        out_shape=jax.ShapeDtypeStruct((R, D), jnp.float32),
    )(h2d, dis, W)


def kernel(x, edge_index, W1, b1, W2, b2):
    n_nodes = x.shape[0]
    src = edge_index[0].astype(jnp.int32)
    dst = edge_index[1].astype(jnp.int32)

    deg = jnp.zeros((n_nodes,), jnp.float32).at[dst].add(1.0) + 1.0
    dis = jax.lax.rsqrt(deg)[:, None]

    outs = []
    for t in range(x.shape[1]):
        h = x[:, t, :]
        for (W, b) in ((W1, b1), (W2, b2)):
            hws = _mm_scaled(h, dis, W)
            seg = jnp.zeros((n_nodes, D), jnp.float32).at[dst].add(hws[src])
            h = jax.nn.relu(dis * (seg + hws) + b)
        outs.append(h)
    return jnp.stack(outs, axis=1)


# speedup vs baseline: 6.1611x; 2.4910x over previous
import functools

import jax
import jax.numpy as jnp
from jax import lax
from jax.experimental import pallas as pl
from jax.experimental.pallas import tpu as pltpu
from jax.experimental.pallas import tpu_sc as plsc

N = 10000
D = 128
T = 8
E = 320000

CHUNK = 64
NBUF = 4
IBLK = 32
TILES = 16
TPC = T // 2
CPT = 320
EPT = CPT * CHUNK
EPAD = EPT * TILES
TRASH = N
ACC_ROWS = 10240
ZROWS = 640
ZB = 16
DEG_IBLK = 16

_mesh = plsc.VectorSubcoreMesh(core_axis_name="c", subcore_axis_name="s", num_cores=2, num_subcores=16)


@functools.partial(
    pl.kernel,
    mesh=_mesh,
    out_type=jax.ShapeDtypeStruct((2, ACC_ROWS, D), jnp.float32),
    scratch_types=[
        pltpu.VMEM((DEG_IBLK, CHUNK), jnp.int32),
        pltpu.VMEM((CHUNK, D), jnp.float32),
        pltpu.VMEM_SHARED((ACC_ROWS, D), jnp.float32),
    ],
)
def _deg_kernel(dstp, ones_in, zeros_in, out, dstb, ones, acc):
    c = lax.axis_index("c")
    s = lax.axis_index("s")

    pltpu.sync_copy(ones_in, ones)
    pltpu.sync_copy(zeros_in, acc.at[pl.ds(s * ZROWS, ZROWS)])
    plsc.subcore_barrier()

    base0 = c * (EPAD // CHUNK // 2) + s * (EPAD // CHUNK // 32)

    def block_body(blk, _):
        pltpu.sync_copy(dstp.at[pl.ds(base0 + blk * DEG_IBLK, DEG_IBLK)], dstb)
        for g in range(DEG_IBLK):
            pltpu.sync_copy(ones, acc.at[dstb.at[g]], add=True)
        return 0

    lax.fori_loop(0, EPAD // CHUNK // 32 // DEG_IBLK, block_body, 0)
    plsc.subcore_barrier()

    pltpu.sync_copy(
        acc.at[pl.ds(s * ZROWS, ZROWS)], out.at[c, pl.ds(s * ZROWS, ZROWS)]
    )


@functools.partial(
    pl.kernel,
    mesh=_mesh,
    out_type=jax.ShapeDtypeStruct((T, ACC_ROWS, D), jnp.float32),
    scratch_types=[
        pltpu.VMEM((IBLK, CHUNK), jnp.int32),
        pltpu.VMEM((IBLK, CHUNK), jnp.int32),
        pltpu.VMEM((NBUF, CHUNK, D), jnp.float32),
        pltpu.VMEM_SHARED((ACC_ROWS, D), jnp.float32),
        pltpu.SemaphoreType.DMA,
        pltpu.SemaphoreType.DMA,
    ],
)
def _segsum_kernel(table, srcp, dstp, zeros_in, out, gidx, dstb, rows, acc, gsem, ssem):
    c = lax.axis_index("c")
    s = lax.axis_index("s")

    def per_t(tt, _):
        t = c * TPC + tt
        off = t * N

        pltpu.sync_copy(zeros_in, acc.at[pl.ds(s * ZROWS, ZROWS)])
        plsc.subcore_barrier()

        def block_body(blk, _):
            base = s * CPT + blk * IBLK
            pltpu.sync_copy(srcp.at[pl.ds(base, IBLK)], gidx)
            pltpu.sync_copy(dstp.at[pl.ds(base, IBLK)], dstb)

            def idx_row(r, _):
                for j in range(CHUNK // 16):
                    v = gidx[r, pl.ds(j * 16, 16)]
                    gidx[r, pl.ds(j * 16, 16)] = v + off
                return 0

            lax.fori_loop(0, IBLK, idx_row, 0)

            for g in range(IBLK // NBUF):
                gcps = []
                for k in range(NBUF):
                    gcps.append(
                        pltpu.async_copy(
                            table.at[gidx.at[g * NBUF + k]], rows.at[k], gsem
                        )
                    )
                scps = []
                for k in range(NBUF):
                    gcps[k].wait()
                    scps.append(
                        pltpu.async_copy(
                            rows.at[k], acc.at[dstb.at[g * NBUF + k]], ssem,
                            add=True,
                        )
                    )
                for cp in scps:
                    cp.wait()
            return 0

        lax.fori_loop(0, CPT // IBLK, block_body, 0)
        plsc.subcore_barrier()

        pltpu.sync_copy(
            acc.at[pl.ds(s * ZROWS, ZROWS)], out.at[t, pl.ds(s * ZROWS, ZROWS)]
        )
        plsc.subcore_barrier()
        return 0

    lax.fori_loop(0, TPC, per_t, 0)


NB = 1000


def _tc1_body(x_ref, deg_ref, w_ref, o_ref):
    dis = lax.rsqrt(deg_ref[0, :, 0:1] + deg_ref[1, :, 0:1] + 1.0)
    for t in range(T):
        hw = jnp.dot(x_ref[:, t, :], w_ref[...], preferred_element_type=jnp.float32)
        o_ref[t] = hw * dis


def _tc2_body(seg_ref, hws_ref, deg_ref, w_ref, b_ref, o_ref):
    dis = lax.rsqrt(deg_ref[0, :, 0:1] + deg_ref[1, :, 0:1] + 1.0)
    h = jax.nn.relu(dis * (seg_ref[0] + hws_ref[0]) + b_ref[...])
    hw = jnp.dot(h, w_ref[...], preferred_element_type=jnp.float32)
    o_ref[0] = hw * dis


def _tc3_body(seg_ref, hws_ref, deg_ref, b_ref, o_ref):
    dis = lax.rsqrt(deg_ref[0, :, 0:1] + deg_ref[1, :, 0:1] + 1.0)
    for t in range(T):
        o_ref[:, t, :] = jax.nn.relu(dis * (seg_ref[t] + hws_ref[t]) + b_ref[...])


def _tc1(x, deg16, W1):
    return pl.pallas_call(
        _tc1_body,
        grid=(N // NB,),
        in_specs=[
            pl.BlockSpec((NB, T, D), lambda n: (n, 0, 0)),
            pl.BlockSpec((2, NB, D), lambda n: (0, n, 0)),
            pl.BlockSpec((D, D), lambda n: (0, 0)),
        ],
        out_specs=pl.BlockSpec((T, NB, D), lambda n: (0, n, 0)),
        out_shape=jax.ShapeDtypeStruct((T, N, D), jnp.float32),
    )(x, deg16, W1)


def _tc2(seg1, hws1, deg16, W2, b1):
    return pl.pallas_call(
        _tc2_body,
        grid=(T, N // NB),
        in_specs=[
            pl.BlockSpec((1, NB, D), lambda t, n: (t, n, 0)),
            pl.BlockSpec((1, NB, D), lambda t, n: (t, n, 0)),
            pl.BlockSpec((2, NB, D), lambda t, n: (0, n, 0)),
            pl.BlockSpec((D, D), lambda t, n: (0, 0)),
            pl.BlockSpec((1, D), lambda t, n: (0, 0)),
        ],
        out_specs=pl.BlockSpec((1, NB, D), lambda t, n: (t, n, 0)),
        out_shape=jax.ShapeDtypeStruct((T, N, D), jnp.float32),
    )(seg1, hws1, deg16, W2, b1)


def _tc3(seg2, hws2, deg16, b2):
    return pl.pallas_call(
        _tc3_body,
        grid=(N // NB,),
        in_specs=[
            pl.BlockSpec((T, NB, D), lambda n: (0, n, 0)),
            pl.BlockSpec((T, NB, D), lambda n: (0, n, 0)),
            pl.BlockSpec((2, NB, D), lambda n: (0, n, 0)),
            pl.BlockSpec((1, D), lambda n: (0, 0)),
        ],
        out_specs=pl.BlockSpec((NB, T, D), lambda n: (n, 0, 0)),
        out_shape=jax.ShapeDtypeStruct((N, T, D), jnp.float32),
    )(seg2, hws2, deg16, b2)


def kernel(x, edge_index, W1, b1, W2, b2):
    src = edge_index[0].astype(jnp.int32)
    dst = edge_index[1].astype(jnp.int32)
    npad = EPAD - E
    srcp = jnp.concatenate([src, jnp.zeros((npad,), jnp.int32)])
    dstp = jnp.concatenate([dst, jnp.full((npad,), TRASH, jnp.int32)])
    srcp = srcp.reshape(EPAD // CHUNK, CHUNK)
    dstp = dstp.reshape(EPAD // CHUNK, CHUNK)

    onesd = jnp.ones((CHUNK, D), jnp.float32)
    zerosd = jnp.zeros((ZROWS, D), jnp.float32)
    deg_p = _deg_kernel(dstp, onesd, zerosd)[:, :N]
    hws1 = _tc1(x, deg_p, W1)
    seg1 = _segsum_kernel(hws1.reshape(T * N, D), srcp, dstp, zerosd)[:, :N]
    hws2 = _tc2(seg1, hws1, deg_p, W2, b1.reshape(1, D))
    seg2 = _segsum_kernel(hws2.reshape(T * N, D), srcp, dstp, zerosd)[:, :N]
    return _tc3(seg2, hws2, deg_p, b2.reshape(1, D))


# speedup vs baseline: 6.1930x; 1.0052x over previous
import functools

import jax
import jax.numpy as jnp
from jax import lax
from jax.experimental import pallas as pl
from jax.experimental.pallas import tpu as pltpu
from jax.experimental.pallas import tpu_sc as plsc

N = 10000
D = 128
T = 8
E = 320000

CHUNK = 64
NBUF = 4
IBLK = 32
TILES = 16
TPC = T // 2
CPT = 320
EPT = CPT * CHUNK
EPAD = EPT * TILES
TRASH = N
ACC_ROWS = 10240
ZROWS = 640
ZB = 64
DEG_IBLK = 16

_mesh = plsc.VectorSubcoreMesh(core_axis_name="c", subcore_axis_name="s", num_cores=2, num_subcores=16)


@functools.partial(
    pl.kernel,
    mesh=_mesh,
    out_type=jax.ShapeDtypeStruct((2, ACC_ROWS, D), jnp.float32),
    scratch_types=[
        pltpu.VMEM((DEG_IBLK, CHUNK), jnp.int32),
        pltpu.VMEM((CHUNK, D), jnp.float32),
        pltpu.VMEM_SHARED((ACC_ROWS, D), jnp.float32),
    ],
)
def _deg_kernel(dstp, ones_in, zeros_in, out, dstb, ones, acc):
    c = lax.axis_index("c")
    s = lax.axis_index("s")

    pltpu.sync_copy(ones_in, ones)
    pltpu.sync_copy(zeros_in, acc.at[pl.ds(s * ZROWS, ZROWS)])
    plsc.subcore_barrier()

    base0 = c * (EPAD // CHUNK // 2) + s * (EPAD // CHUNK // 32)

    def block_body(blk, _):
        pltpu.sync_copy(dstp.at[pl.ds(base0 + blk * DEG_IBLK, DEG_IBLK)], dstb)
        for g in range(DEG_IBLK):
            pltpu.sync_copy(ones, acc.at[dstb.at[g]], add=True)
        return 0

    lax.fori_loop(0, EPAD // CHUNK // 32 // DEG_IBLK, block_body, 0)
    plsc.subcore_barrier()

    pltpu.sync_copy(
        acc.at[pl.ds(s * ZROWS, ZROWS)], out.at[c, pl.ds(s * ZROWS, ZROWS)]
    )


@functools.partial(
    pl.kernel,
    mesh=_mesh,
    out_type=jax.ShapeDtypeStruct((T, ACC_ROWS, D), jnp.float32),
    scratch_types=[
        pltpu.VMEM((IBLK, CHUNK), jnp.int32),
        pltpu.VMEM((IBLK, CHUNK), jnp.int32),
        pltpu.VMEM((NBUF, CHUNK, D), jnp.float32),
        pltpu.VMEM((ZB, D), jnp.float32),
        pltpu.VMEM_SHARED((ACC_ROWS, D), jnp.float32),
        pltpu.SemaphoreType.DMA,
        pltpu.SemaphoreType.DMA,
    ],
)
def _segsum_kernel(table, srcp, dstp, zeros_in, out, gidx, dstb, rows, zb, acc, gsem, ssem):
    c = lax.axis_index("c")
    s = lax.axis_index("s")

    pltpu.sync_copy(zeros_in.at[pl.ds(0, ZB)], zb)

    def per_t(tt, _):
        t = c * TPC + tt
        off = t * N

        def zero_chunk(k, _):
            pltpu.sync_copy(zb, acc.at[pl.ds(s * ZROWS + k * ZB, ZB)])
            return 0

        lax.fori_loop(0, ZROWS // ZB, zero_chunk, 0)
        plsc.subcore_barrier()

        def block_body(blk, _):
            base = s * CPT + blk * IBLK
            pltpu.sync_copy(srcp.at[pl.ds(base, IBLK)], gidx)
            pltpu.sync_copy(dstp.at[pl.ds(base, IBLK)], dstb)

            def idx_row(r, _):
                for j in range(CHUNK // 16):
                    v = gidx[r, pl.ds(j * 16, 16)]
                    gidx[r, pl.ds(j * 16, 16)] = v + off
                return 0

            lax.fori_loop(0, IBLK, idx_row, 0)

            for g in range(IBLK // NBUF):
                gcps = []
                for k in range(NBUF):
                    gcps.append(
                        pltpu.async_copy(
                            table.at[gidx.at[g * NBUF + k]], rows.at[k], gsem
                        )
                    )
                scps = []
                for k in range(NBUF):
                    gcps[k].wait()
                    scps.append(
                        pltpu.async_copy(
                            rows.at[k], acc.at[dstb.at[g * NBUF + k]], ssem,
                            add=True,
                        )
                    )
                for cp in scps:
                    cp.wait()
            return 0

        lax.fori_loop(0, CPT // IBLK, block_body, 0)
        plsc.subcore_barrier()

        pltpu.sync_copy(
            acc.at[pl.ds(s * ZROWS, ZROWS)], out.at[t, pl.ds(s * ZROWS, ZROWS)]
        )
        plsc.subcore_barrier()
        return 0

    lax.fori_loop(0, TPC, per_t, 0)


NB = 1000


def _tc1_body(x_ref, deg_ref, w_ref, o_ref):
    dis = lax.rsqrt(deg_ref[0, :, 0:1] + deg_ref[1, :, 0:1] + 1.0)
    for t in range(T):
        hw = jnp.dot(x_ref[:, t, :], w_ref[...], preferred_element_type=jnp.float32)
        o_ref[t] = hw * dis


def _tc2_body(seg_ref, hws_ref, deg_ref, w_ref, b_ref, o_ref):
    dis = lax.rsqrt(deg_ref[0, :, 0:1] + deg_ref[1, :, 0:1] + 1.0)
    h = jax.nn.relu(dis * (seg_ref[0] + hws_ref[0]) + b_ref[...])
    hw = jnp.dot(h, w_ref[...], preferred_element_type=jnp.float32)
    o_ref[0] = hw * dis


def _tc3_body(seg_ref, hws_ref, deg_ref, b_ref, o_ref):
    dis = lax.rsqrt(deg_ref[0, :, 0:1] + deg_ref[1, :, 0:1] + 1.0)
    for t in range(T):
        o_ref[:, t, :] = jax.nn.relu(dis * (seg_ref[t] + hws_ref[t]) + b_ref[...])


def _tc1(x, deg16, W1):
    return pl.pallas_call(
        _tc1_body,
        grid=(N // NB,),
        in_specs=[
            pl.BlockSpec((NB, T, D), lambda n: (n, 0, 0)),
            pl.BlockSpec((2, NB, D), lambda n: (0, n, 0)),
            pl.BlockSpec((D, D), lambda n: (0, 0)),
        ],
        out_specs=pl.BlockSpec((T, NB, D), lambda n: (0, n, 0)),
        out_shape=jax.ShapeDtypeStruct((T, N, D), jnp.float32),
    )(x, deg16, W1)


def _tc2(seg1, hws1, deg16, W2, b1):
    return pl.pallas_call(
        _tc2_body,
        grid=(T, N // NB),
        in_specs=[
            pl.BlockSpec((1, NB, D), lambda t, n: (t, n, 0)),
            pl.BlockSpec((1, NB, D), lambda t, n: (t, n, 0)),
            pl.BlockSpec((2, NB, D), lambda t, n: (0, n, 0)),
            pl.BlockSpec((D, D), lambda t, n: (0, 0)),
            pl.BlockSpec((1, D), lambda t, n: (0, 0)),
        ],
        out_specs=pl.BlockSpec((1, NB, D), lambda t, n: (t, n, 0)),
        out_shape=jax.ShapeDtypeStruct((T, N, D), jnp.float32),
    )(seg1, hws1, deg16, W2, b1)


def _tc3(seg2, hws2, deg16, b2):
    return pl.pallas_call(
        _tc3_body,
        grid=(N // NB,),
        in_specs=[
            pl.BlockSpec((T, NB, D), lambda n: (0, n, 0)),
            pl.BlockSpec((T, NB, D), lambda n: (0, n, 0)),
            pl.BlockSpec((2, NB, D), lambda n: (0, n, 0)),
            pl.BlockSpec((1, D), lambda n: (0, 0)),
        ],
        out_specs=pl.BlockSpec((NB, T, D), lambda n: (n, 0, 0)),
        out_shape=jax.ShapeDtypeStruct((N, T, D), jnp.float32),
    )(seg2, hws2, deg16, b2)


def kernel(x, edge_index, W1, b1, W2, b2):
    src = edge_index[0].astype(jnp.int32)
    dst = edge_index[1].astype(jnp.int32)
    npad = EPAD - E
    srcp = jnp.concatenate([src, jnp.zeros((npad,), jnp.int32)])
    dstp = jnp.concatenate([dst, jnp.full((npad,), TRASH, jnp.int32)])
    srcp = srcp.reshape(EPAD // CHUNK, CHUNK)
    dstp = dstp.reshape(EPAD // CHUNK, CHUNK)

    onesd = jnp.ones((CHUNK, D), jnp.float32)
    zerosd = jnp.zeros((ZROWS, D), jnp.float32)
    deg_p = _deg_kernel(dstp, onesd, zerosd)[:, :N]
    hws1 = _tc1(x, deg_p, W1)
    seg1 = _segsum_kernel(hws1.reshape(T * N, D), srcp, dstp, zerosd)[:, :N]
    hws2 = _tc2(seg1, hws1, deg_p, W2, b1.reshape(1, D))
    seg2 = _segsum_kernel(hws2.reshape(T * N, D), srcp, dstp, zerosd)[:, :N]
    return _tc3(seg2, hws2, deg_p, b2.reshape(1, D))
